# baseline (speedup 1.0000x reference)
import functools

import jax
import jax.numpy as jnp
from jax import lax
from jax.experimental import pallas as pl
from jax.experimental.pallas import tpu as pltpu
from jax.experimental.pallas import tpu_sc as plsc

D_MODEL = 64
BATCH = 1024
SEQ_LEN = 200
NC, NS, LANES = 2, 16, 16
NW = NC * NS
BBLK = 128
NJ = BATCH // BBLK
UNITS = SEQ_LEN * NJ
UPW = UNITS // NW


def _pos_encoding(seq_len, d_model):
    i_model = jnp.repeat(jnp.arange(d_model // 2), 2)
    div_term = jnp.exp(
        i_model.astype(jnp.float32) / d_model * jnp.log(jnp.float32(10000.0))
    )
    pos = jnp.arange(seq_len, dtype=jnp.float32)[:, None] / div_term
    even = (jnp.arange(d_model) % 2) == 0
    return jnp.where(even[None, :], jnp.sin(pos), jnp.cos(pos))


def _body(xt_ref, tab_ref, pe_ref, out_ref, pe_v, idxs, cols, gbufs, obufs, sems):
    gsems, osems = sems[:2], sems[2:]
    wid = lax.axis_index("s") * NC + lax.axis_index("c")
    u0 = wid * UPW
    pltpu.sync_copy(pe_ref, pe_v)

    iota = lax.iota(jnp.int32, LANES)
    rowv = [iota + 16 * jb for jb in range(8)]

    def stage_idx(u, p):
        l = u // NJ
        j = u % NJ
        pltpu.sync_copy(xt_ref.at[l, pl.ds(j * BBLK, BBLK)], idxs.at[p])
        for jb in range(8):
            sl = pl.ds(jb * LANES, LANES)
            v = idxs[p, sl]
            idxs[p, sl] = v >> 1
            cols[p, sl] = (v & 1) * D_MODEL

    def fire_gather(p):
        pltpu.async_copy(tab_ref.at[idxs.at[p]], gbufs.at[p], gsems[p])

    def wait_gather(p):
        pltpu.make_async_copy(
            tab_ref.at[idxs.at[p]], gbufs.at[p], gsems[p]
        ).wait()

    def compute(u, p):
        l = u // NJ
        cb = [cols[p, pl.ds(jb * LANES, LANES)] for jb in range(8)]
        g = gbufs.at[p]
        ob = obufs.at[p]
        pbase = l * D_MODEL

        @pl.loop(0, D_MODEL)
        def _d(d):
            peb = jnp.broadcast_to(pe_v[pl.ds(pbase + d, LANES)][0], (LANES,))
            for jb in range(8):
                vals = plsc.load_gather(g, [rowv[jb], cb[jb] + d])
                ob[d, pl.ds(jb * LANES, LANES)] = vals + peb

    def fire_out(u, p):
        l = u // NJ
        j = u % NJ
        pltpu.async_copy(
            obufs.at[p], out_ref.at[l, :, pl.ds(j * BBLK, BBLK)], osems[p]
        )

    def wait_out(p):
        pltpu.make_async_copy(
            obufs.at[p], out_ref.at[0, :, pl.ds(0, BBLK)], osems[p]
        ).wait()

    stage_idx(u0, 0)
    fire_gather(0)
    stage_idx(u0 + 1, 1)
    fire_gather(1)

    wait_gather(0)
    compute(u0, 0)
    fire_out(u0, 0)
    stage_idx(u0 + 2, 0)
    fire_gather(0)

    wait_gather(1)
    compute(u0 + 1, 1)
    fire_out(u0 + 1, 1)
    stage_idx(u0 + 3, 1)
    fire_gather(1)

    @pl.loop(1, UPW // 2)
    def _s(s):
        for p in range(2):
            t = 2 * s + p
            u = u0 + t
            wait_gather(p)
            wait_out(p)
            compute(u, p)
            fire_out(u, p)
            u_next = u0 + jnp.minimum(t + 2, UPW - 1)
            stage_idx(u_next, p)
            fire_gather(p)

    wait_gather(0)
    wait_gather(1)
    wait_out(0)
    wait_out(1)


@functools.partial(jax.jit, static_argnums=())
def _emb_lookup(xt, tab2, pe):
    mesh = plsc.VectorSubcoreMesh(
        core_axis_name="c", subcore_axis_name="s", num_cores=NC, num_subcores=NS
    )
    f = pl.kernel(
        _body,
        out_type=jax.ShapeDtypeStruct((SEQ_LEN, D_MODEL, BATCH), jnp.float32),
        mesh=mesh,
        scratch_types=[
            pltpu.VMEM((SEQ_LEN * D_MODEL + LANES,), jnp.float32),
            pltpu.VMEM((2, BBLK), jnp.int32),
            pltpu.VMEM((2, BBLK), jnp.int32),
            pltpu.VMEM((2, BBLK, 2 * D_MODEL), jnp.float32),
            pltpu.VMEM((2, D_MODEL, BBLK), jnp.float32),
            [pltpu.SemaphoreType.DMA] * 4,
        ],
        compiler_params=pltpu.CompilerParams(use_tc_tiling_on_sc=True, needs_layout_passes=False),
    )
    return f(xt, tab2, pe)


def kernel(x, emb_weight):
    pe = jnp.concatenate(
        [_pos_encoding(SEQ_LEN, D_MODEL).reshape(-1), jnp.zeros((LANES,), jnp.float32)]
    )
    xt = x.T
    tab2 = emb_weight.reshape(500000, 2 * D_MODEL)
    p_out = _emb_lookup(xt, tab2, pe)
    return p_out.transpose(2, 0, 1)


# speedup vs baseline: 1.1666x; 1.1666x over previous
import functools

import jax
import jax.numpy as jnp
from jax import lax
from jax.experimental import pallas as pl
from jax.experimental.pallas import tpu as pltpu
from jax.experimental.pallas import tpu_sc as plsc

D_MODEL = 64
BATCH = 1024
SEQ_LEN = 200
NC, NS, LANES = 2, 16, 16
NW = NC * NS
RPW = BATCH // NW
NBUF = 4


def _pos_encoding(seq_len, d_model):
    i_model = jnp.repeat(jnp.arange(d_model // 2), 2)
    div_term = jnp.exp(
        i_model.astype(jnp.float32) / d_model * jnp.log(jnp.float32(10000.0))
    )
    pos = jnp.arange(seq_len, dtype=jnp.float32)[:, None] / div_term
    even = (jnp.arange(d_model) % 2) == 0
    return jnp.where(even[None, :], jnp.sin(pos), jnp.cos(pos))


def _body(x_ref, tab_ref, pe_ref, out_ref, idx_v, bufs, pe_v, sems):
    gsems, osems = sems[:NBUF], sems[NBUF:]
    wid = lax.axis_index("s") * NC + lax.axis_index("c")
    b0 = wid * RPW
    pltpu.sync_copy(x_ref.at[pl.ds(b0, RPW), :], idx_v)
    pltpu.sync_copy(pe_ref, pe_v)

    @pl.loop(0, RPW // NBUF)
    def _group(t):
        r0 = t * NBUF
        gds = [
            pltpu.async_copy(tab_ref.at[idx_v.at[r0 + k]], bufs.at[k], gsems[k])
            for k in range(NBUF)
        ]
        ods = []
        for k in range(NBUF):
            gds[k].wait()
            buf = bufs.at[k]

            @pl.loop(0, SEQ_LEN, unroll=8)
            def _row(r):
                for q in range(D_MODEL // LANES):
                    sl = pl.ds(q * LANES, LANES)
                    buf[r, sl] = buf[r, sl] + pe_v[r, sl]

            ods.append(
                pltpu.async_copy(buf, out_ref.at[b0 + r0 + k], osems[k])
            )
        for d in ods:
            d.wait()


@functools.partial(jax.jit, static_argnums=())
def _emb_lookup(x, emb_weight, pe):
    mesh = plsc.VectorSubcoreMesh(
        core_axis_name="c", subcore_axis_name="s", num_cores=NC, num_subcores=NS
    )
    f = pl.kernel(
        _body,
        out_type=jax.ShapeDtypeStruct((BATCH, SEQ_LEN, D_MODEL), jnp.float32),
        mesh=mesh,
        scratch_types=[
            pltpu.VMEM((RPW, SEQ_LEN), jnp.int32),
            pltpu.VMEM((NBUF, SEQ_LEN, D_MODEL), jnp.float32),
            pltpu.VMEM((SEQ_LEN, D_MODEL), jnp.float32),
            [pltpu.SemaphoreType.DMA] * (2 * NBUF),
        ],
        compiler_params=pltpu.CompilerParams(use_tc_tiling_on_sc=False),
    )
    return f(x, emb_weight, pe)


def kernel(x, emb_weight):
    pe = _pos_encoding(SEQ_LEN, D_MODEL)
    return _emb_lookup(x, emb_weight, pe)
